# baseline (speedup 1.0000x reference)
import functools

import jax
import jax.numpy as jnp
from jax.experimental import pallas as pl
from jax.experimental.pallas import tpu as pltpu

_N_ACTIONS = 64
_HW = 256
_BIAS_ROW0 = 1280


def _ac_kernel(n_actions, x_ref, w_ref, b_ref, bo_ref, pol_ref, val_ref):
    w1 = w_ref[0:256, :]
    w2a = w_ref[256:512, 0:256]
    w2c = w_ref[512:768, 256:512]
    woa = w_ref[768:1024, 0:128]
    wv = w_ref[1024:1280, 0:128]

    b1 = b_ref[0:1, :]
    b2 = b_ref[8:9, :]

    x = x_ref[...].astype(jnp.bfloat16)

    h1 = jnp.tanh(jnp.dot(x, w1, preferred_element_type=jnp.float32) + b1)
    h1 = h1.astype(jnp.bfloat16)

    h2a = jnp.tanh(jnp.dot(h1[:, 0:256], w2a,
                           preferred_element_type=jnp.float32) + b2[:, 0:256])
    h2c = jnp.tanh(jnp.dot(h1[:, 256:512], w2c,
                           preferred_element_type=jnp.float32) + b2[:, 256:512])

    dn = (((0,), (1,)), ((), ()))
    oT = (jax.lax.dot_general(woa, h2a.astype(jnp.bfloat16), dn,
                              preferred_element_type=jnp.float32)
          + jax.lax.dot_general(wv, h2c.astype(jnp.bfloat16), dn,
                                preferred_element_type=jnp.float32)
          + bo_ref[...])

    lg = oT[0:n_actions, :]
    m = jnp.max(lg, axis=0, keepdims=True)
    e = jnp.exp(lg - m)
    inv = 1.0 / jnp.sum(e, axis=0, keepdims=True)
    pol_ref[...] = e * inv

    val_ref[...] = oT[n_actions, :]


def kernel(states, slab):
    B, in_dim = states.shape
    n_actions = _N_ACTIONS

    w_bf16 = slab[:_BIAS_ROW0].astype(jnp.bfloat16)
    biases = slab[_BIAS_ROW0:]
    bo_col = slab[1296:1297, 0:128].T

    block_b = min(B, 8192)
    grid = (pl.cdiv(B, block_b),)

    polT, valT = pl.pallas_call(
        functools.partial(_ac_kernel, n_actions),
        out_shape=(
            jax.ShapeDtypeStruct((n_actions, B), jnp.float32),
            jax.ShapeDtypeStruct((B,), jnp.float32),
        ),
        grid=grid,
        in_specs=[
            pl.BlockSpec((block_b, in_dim), lambda i: (i, 0)),
            pl.BlockSpec(w_bf16.shape, lambda i: (0, 0)),
            pl.BlockSpec(biases.shape, lambda i: (0, 0)),
            pl.BlockSpec(bo_col.shape, lambda i: (0, 0)),
        ],
        out_specs=(
            pl.BlockSpec((n_actions, block_b), lambda i: (0, i)),
            pl.BlockSpec((block_b,), lambda i: (i,)),
        ),
        compiler_params=pltpu.CompilerParams(
            dimension_semantics=("parallel",)),
    )(states, w_bf16, biases, bo_col)

    return polT.T, valT.reshape(B, 1)


# speedup vs baseline: 1.0113x; 1.0113x over previous
import functools

import jax
import jax.numpy as jnp
from jax.experimental import pallas as pl
from jax.experimental.pallas import tpu as pltpu

_N_ACTIONS = 64
_HW = 256
_BIAS_ROW0 = 1280


def _ac_kernel(n_actions, x_ref, w_ref, b_ref, bo_ref, pol_ref, val_ref):
    w1 = w_ref[0:256, :]
    w2a = w_ref[256:512, 0:256]
    w2c = w_ref[512:768, 256:512]
    woa = w_ref[768:1024, 0:128]
    wv = w_ref[1024:1280, 0:128]

    b1 = b_ref[1280:1281, :]
    b2 = b_ref[1288:1289, :]

    x = x_ref[...].astype(jnp.bfloat16)

    h1 = jnp.tanh(jnp.dot(x, w1, preferred_element_type=jnp.float32) + b1)
    h1 = h1.astype(jnp.bfloat16)

    h2a = jnp.tanh(jnp.dot(h1[:, 0:256], w2a,
                           preferred_element_type=jnp.float32) + b2[:, 0:256])
    h2c = jnp.tanh(jnp.dot(h1[:, 256:512], w2c,
                           preferred_element_type=jnp.float32) + b2[:, 256:512])

    dn = (((0,), (1,)), ((), ()))
    oT = (jax.lax.dot_general(woa, h2a.astype(jnp.bfloat16), dn,
                              preferred_element_type=jnp.float32)
          + jax.lax.dot_general(wv, h2c.astype(jnp.bfloat16), dn,
                                preferred_element_type=jnp.float32)
          + bo_ref[...])

    lg = oT[0:n_actions, :]
    m = jnp.max(lg, axis=0, keepdims=True)
    e = jnp.exp(lg - m)
    inv = 1.0 / jnp.sum(e, axis=0, keepdims=True)
    pol_ref[...] = e * inv

    val_ref[...] = oT[n_actions, :]


def kernel(states, slab):
    B, in_dim = states.shape
    n_actions = _N_ACTIONS

    w_bf16 = slab[:_BIAS_ROW0].astype(jnp.bfloat16)
    bo_col = slab[1296:1297, 0:128].T

    block_b = min(B, 8192)
    grid = (pl.cdiv(B, block_b),)

    polT, valT = pl.pallas_call(
        functools.partial(_ac_kernel, n_actions),
        out_shape=(
            jax.ShapeDtypeStruct((n_actions, B), jnp.float32),
            jax.ShapeDtypeStruct((B,), jnp.float32),
        ),
        grid=grid,
        in_specs=[
            pl.BlockSpec((block_b, in_dim), lambda i: (i, 0)),
            pl.BlockSpec(w_bf16.shape, lambda i: (0, 0)),
            pl.BlockSpec(slab.shape, lambda i: (0, 0)),
            pl.BlockSpec(bo_col.shape, lambda i: (0, 0)),
        ],
        out_specs=(
            pl.BlockSpec((n_actions, block_b), lambda i: (0, i)),
            pl.BlockSpec((block_b,), lambda i: (i,)),
        ),
        compiler_params=pltpu.CompilerParams(
            dimension_semantics=("parallel",)),
    )(states, w_bf16, slab, bo_col)

    return polT.T, valT.reshape(B, 1)


# speedup vs baseline: 1.0149x; 1.0035x over previous
import functools

import jax
import jax.numpy as jnp
from jax.experimental import pallas as pl
from jax.experimental.pallas import tpu as pltpu

_N_ACTIONS = 64
_BIAS_ROW0 = 1280


def _ac_kernel(n_actions, x_ref, w_ref, b_ref, bo_ref, pol_ref, val_ref):
    w1 = w_ref[0:256, :]
    w2a = w_ref[256:512, 0:256]
    w2c = w_ref[512:768, 256:512]
    woa = w_ref[768:1024, 0:128]
    wv = w_ref[1024:1280, 0:128]

    b1 = b_ref[1280:1281, :]
    b2 = b_ref[1288:1289, :]

    x = x_ref[...].astype(jnp.bfloat16)

    h1 = jnp.tanh(jnp.dot(x, w1, preferred_element_type=jnp.float32) + b1)
    h1 = h1.astype(jnp.bfloat16)

    h2a = jnp.tanh(jnp.dot(h1[:, 0:256], w2a,
                           preferred_element_type=jnp.float32) + b2[:, 0:256])
    h2c = jnp.tanh(jnp.dot(h1[:, 256:512], w2c,
                           preferred_element_type=jnp.float32) + b2[:, 256:512])

    dn = (((0,), (1,)), ((), ()))
    oT = (jax.lax.dot_general(woa, h2a.astype(jnp.bfloat16), dn,
                              preferred_element_type=jnp.float32)
          + jax.lax.dot_general(wv, h2c.astype(jnp.bfloat16), dn,
                                preferred_element_type=jnp.float32)
          + bo_ref[...])

    lg = oT[0:n_actions, :]
    m = jnp.max(lg, axis=0, keepdims=True)
    e = jnp.exp(lg - m)
    inv = 1.0 / jnp.sum(e, axis=0, keepdims=True)
    pol_ref[...] = e * inv

    val_ref[...] = oT[n_actions, :]


def kernel(states, slab):
    B, in_dim = states.shape
    n_actions = _N_ACTIONS

    w_bf16 = slab[:_BIAS_ROW0].astype(jnp.bfloat16)
    bo_col = slab[1296:1297, 0:128].T

    block_b = min(B, 8192)
    grid = (pl.cdiv(B, block_b),)

    polT, valT = pl.pallas_call(
        functools.partial(_ac_kernel, n_actions),
        out_shape=(
            jax.ShapeDtypeStruct((n_actions, B), jnp.float32),
            jax.ShapeDtypeStruct((B,), jnp.float32),
        ),
        grid=grid,
        in_specs=[
            pl.BlockSpec((block_b, in_dim), lambda i: (i, 0)),
            pl.BlockSpec(w_bf16.shape, lambda i: (0, 0)),
            pl.BlockSpec(slab.shape, lambda i: (0, 0)),
            pl.BlockSpec(bo_col.shape, lambda i: (0, 0)),
        ],
        out_specs=(
            pl.BlockSpec((n_actions, block_b), lambda i: (0, i)),
            pl.BlockSpec((block_b,), lambda i: (i,)),
        ),
        compiler_params=pltpu.CompilerParams(
            dimension_semantics=("parallel",)),
    )(states, w_bf16, slab, bo_col)

    return polT.T, valT.reshape(B, 1)
